# baseline (speedup 1.0000x reference)
import functools

import jax
import jax.numpy as jnp
from jax import lax
from jax.experimental import pallas as pl
from jax.experimental.pallas import tpu as pltpu
from jax.experimental.pallas import tpu_sc as plsc

VOCAB = 1000000
EMBED_DIM = 32
PAD_DIM = 128
BATCH = 16384
HIST_LEN = 50

NC = 2
NS = 16
NW = NC * NS

B_PER_W = BATCH // NW
CB = 2
N_CHUNKS = B_PER_W // CB
LANES = 16


def _gather_kernel(idx_hbm, table_hbm, out_hbm,
                   idx_v, slab0, slab1, row0_, row1_, gs0, gs1, ws0, ws1):
    wid = lax.axis_index("s") * NC + lax.axis_index("c")
    b0 = wid * B_PER_W

    pltpu.sync_copy(idx_hbm.at[pl.ds(b0, B_PER_W)], idx_v)

    slabs = (slab0, slab1)
    rows = (row0_, row1_)
    gsems = (gs0, gs1)
    wsems = (ws0, ws1)

    def fire(g, b):
        for s in range(CB):
            pltpu.async_copy(
                table_hbm.at[idx_v.at[g * CB + s]],
                slabs[b].at[s],
                gsems[b],
            )

    def drain_gather(b):
        for s in range(CB):
            pltpu.make_async_copy(
                table_hbm.at[idx_v.at[0]], slabs[b].at[s], gsems[b]
            ).wait()

    def extract(b):
        def t_body(t, carry):
            for s in range(CB):
                for h in range(EMBED_DIM // LANES):
                    rows[b][s, t, pl.ds(h * LANES, LANES)] = (
                        slabs[b][s, t, pl.ds(h * LANES, LANES)]
                    )
            return carry
        lax.fori_loop(0, HIST_LEN, t_body, 0)

    def writeback(g, b):
        pltpu.async_copy(
            rows[b], out_hbm.at[pl.ds(b0 + g * CB, CB)], wsems[b]
        )

    def drain_wb(b):
        pltpu.make_async_copy(
            rows[b], out_hbm.at[pl.ds(0, CB)], wsems[b]
        ).wait()

    fire(0, 0)
    fire(1, 1)

    def pair_body(t, carry):
        g = 2 * t
        drain_gather(0)
        extract(0)
        writeback(g, 0)
        drain_gather(1)
        extract(1)
        writeback(g + 1, 1)

        @pl.when(g + 2 < N_CHUNKS)
        def _fire0():
            drain_wb(0)
            fire(g + 2, 0)

        @pl.when(g + 3 < N_CHUNKS)
        def _fire1():
            drain_wb(1)
            fire(g + 3, 1)

        return carry

    lax.fori_loop(0, N_CHUNKS // 2, pair_body, 0)
    drain_wb(0)
    drain_wb(1)


@jax.jit
def _embedding_gather(idx, table_pad):
    mesh = plsc.VectorSubcoreMesh(core_axis_name="c", subcore_axis_name="s")
    f = functools.partial(
        pl.kernel,
        mesh=mesh,
        out_type=jax.ShapeDtypeStruct((BATCH, HIST_LEN, EMBED_DIM),
                                      jnp.float32),
        scratch_types=[
            pltpu.VMEM((B_PER_W, HIST_LEN), jnp.int32),
            pltpu.VMEM((CB, HIST_LEN, PAD_DIM), jnp.float32),
            pltpu.VMEM((CB, HIST_LEN, PAD_DIM), jnp.float32),
            pltpu.VMEM((CB, HIST_LEN, EMBED_DIM), jnp.float32),
            pltpu.VMEM((CB, HIST_LEN, EMBED_DIM), jnp.float32),
            pltpu.SemaphoreType.DMA,
            pltpu.SemaphoreType.DMA,
            pltpu.SemaphoreType.DMA,
            pltpu.SemaphoreType.DMA,
        ],
    )(_gather_kernel)
    return f(idx, table_pad)


def kernel(input_, table_ids):
    table_pad = jnp.pad(table_ids, ((0, 0), (0, PAD_DIM - EMBED_DIM)))
    return _embedding_gather(input_.astype(jnp.int32), table_pad)


# speedup vs baseline: 1.0760x; 1.0760x over previous
import functools

import jax
import jax.numpy as jnp
from jax import lax
from jax.experimental import pallas as pl
from jax.experimental.pallas import tpu as pltpu
from jax.experimental.pallas import tpu_sc as plsc

VOCAB = 1000000
EMBED_DIM = 32
BATCH = 16384
HIST_LEN = 50

NC = 2
NS = 16
NW = NC * NS

B_PER_W = BATCH // NW
CB = 8
N_CHUNKS = B_PER_W // CB


def _gather_kernel(idx_hbm, table_hbm, out_hbm,
                   idx_v, buf0, buf1, gs0, gs1, ws0, ws1):
    wid = lax.axis_index("s") * NC + lax.axis_index("c")
    b0 = wid * B_PER_W

    pltpu.sync_copy(idx_hbm.at[pl.ds(b0, B_PER_W)], idx_v)

    bufs = (buf0, buf1)
    gsems = (gs0, gs1)
    wsems = (ws0, ws1)

    def fire(g, b):
        for s in range(CB):
            pltpu.async_copy(
                table_hbm.at[idx_v.at[g * CB + s]],
                bufs[b].at[s],
                gsems[b],
            )

    def drain_gather(b):
        pltpu.make_async_copy(
            out_hbm.at[pl.ds(0, CB)], bufs[b], gsems[b]
        ).wait()

    def writeback(g, b):
        pltpu.async_copy(
            bufs[b], out_hbm.at[pl.ds(b0 + g * CB, CB)], wsems[b]
        )

    def drain_wb(b):
        pltpu.make_async_copy(
            bufs[b], out_hbm.at[pl.ds(0, CB)], wsems[b]
        ).wait()

    fire(0, 0)
    fire(1, 1)

    def pair_body(t, carry):
        g = 2 * t
        drain_gather(0)
        writeback(g, 0)
        drain_gather(1)
        writeback(g + 1, 1)

        @pl.when(g + 2 < N_CHUNKS)
        def _fire0():
            drain_wb(0)
            fire(g + 2, 0)

        @pl.when(g + 3 < N_CHUNKS)
        def _fire1():
            drain_wb(1)
            fire(g + 3, 1)

        return carry

    lax.fori_loop(0, N_CHUNKS // 2, pair_body, 0)
    drain_wb(0)
    drain_wb(1)


@jax.jit
def _embedding_gather(idx, table):
    mesh = plsc.VectorSubcoreMesh(core_axis_name="c", subcore_axis_name="s")
    f = functools.partial(
        pl.kernel,
        mesh=mesh,
        out_type=jax.ShapeDtypeStruct((BATCH, HIST_LEN, EMBED_DIM),
                                      jnp.float32),
        scratch_types=[
            pltpu.VMEM((B_PER_W, HIST_LEN), jnp.int32),
            pltpu.VMEM((CB, HIST_LEN, EMBED_DIM), jnp.float32),
            pltpu.VMEM((CB, HIST_LEN, EMBED_DIM), jnp.float32),
            pltpu.SemaphoreType.DMA,
            pltpu.SemaphoreType.DMA,
            pltpu.SemaphoreType.DMA,
            pltpu.SemaphoreType.DMA,
        ],
        compiler_params=pltpu.CompilerParams(use_tc_tiling_on_sc=False),
    )(_gather_kernel)
    return f(idx, table)


def kernel(input_, table_ids):
    return _embedding_gather(input_.astype(jnp.int32), table_ids)
